# Initial kernel scaffold; baseline (speedup 1.0000x reference)
import functools

import jax
import jax.numpy as jnp
from jax import lax
from jax.experimental import pallas as pl
from jax.experimental.pallas import tpu as pltpu
from jax.experimental.pallas import tpu_sc as plsc

ALPHA = 0.1
K = 3
NC = 2
NS = 16
NW = NC * NS
B = 128


def _sc_degree_kernel(np_, nb):
  ch = np_ // NS
  mesh = plsc.VectorSubcoreMesh(core_axis_name="c", subcore_axis_name="s")

  @functools.partial(
      pl.kernel,
      out_type=jax.ShapeDtypeStruct((NC, np_, 16), jnp.float32),
      mesh=mesh,
      scratch_types=[
          pltpu.VMEM((nb, B), jnp.int32),
          pltpu.VMEM((B, 16), jnp.float32),
          pltpu.VMEM((ch, 16), jnp.float32),
          pltpu.VMEM_SHARED((np_, 16), jnp.float32),
          pltpu.SemaphoreType.DMA,
      ],
  )
  def k(dst_hbm, ones_hbm, zeros_hbm, out_hbm, idx_v, ones_v, zbuf_v,
        accum, sem):
    c = lax.axis_index("c")
    s = lax.axis_index("s")
    wid = s * NC + c
    pltpu.sync_copy(zeros_hbm, zbuf_v)
    pltpu.sync_copy(zbuf_v, accum.at[pl.ds(s * ch, ch)])
    pltpu.sync_copy(ones_hbm, ones_v)
    pltpu.sync_copy(dst_hbm.at[wid], idx_v)
    plsc.subcore_barrier()

    def body(j, carry):
      pltpu.sync_copy(ones_v, accum.at[idx_v.at[j]], add=True)
      return carry

    lax.fori_loop(0, nb, body, 0)
    plsc.subcore_barrier()
    pltpu.sync_copy(accum.at[pl.ds(s * ch, ch)], zbuf_v)
    pltpu.sync_copy(zbuf_v, out_hbm.at[c, pl.ds(s * ch, ch)])

  return k


def _sc_hop_kernel(np_, d, nb):
  ch = np_ // NS
  zr = ch // 2
  mesh = plsc.VectorSubcoreMesh(core_axis_name="c", subcore_axis_name="s")

  @functools.partial(
      pl.kernel,
      out_type=jax.ShapeDtypeStruct((NC, np_, d), jnp.float32),
      mesh=mesh,
      scratch_types=[
          pltpu.VMEM((nb, B), jnp.int32),
          pltpu.VMEM((nb, B), jnp.int32),
          pltpu.VMEM((B, d), jnp.float32),
          pltpu.VMEM((zr, d), jnp.float32),
          pltpu.VMEM_SHARED((np_, d), jnp.float32),
          pltpu.SemaphoreType.DMA,
      ],
  )
  def k(y_hbm, src_hbm, dst_hbm, zeros_hbm, out_hbm, src_v, dst_v, rows_v,
        zbuf_v, accum, sem):
    c = lax.axis_index("c")
    s = lax.axis_index("s")
    wid = s * NC + c
    pltpu.sync_copy(zeros_hbm, zbuf_v)
    pltpu.sync_copy(zbuf_v, accum.at[pl.ds(s * ch, zr)])
    pltpu.sync_copy(zbuf_v, accum.at[pl.ds(s * ch + zr, zr)])
    pltpu.sync_copy(src_hbm.at[wid], src_v)
    pltpu.sync_copy(dst_hbm.at[wid], dst_v)
    plsc.subcore_barrier()

    def body(j, carry):
      pltpu.async_copy(y_hbm.at[src_v.at[j]], rows_v, sem).wait()
      pltpu.sync_copy(rows_v, accum.at[dst_v.at[j]], add=True)
      return carry

    lax.fori_loop(0, nb, body, 0)
    plsc.subcore_barrier()
    for half in range(2):
      pltpu.sync_copy(accum.at[pl.ds(s * ch + half * zr, zr)], zbuf_v)
      pltpu.sync_copy(zbuf_v, out_hbm.at[c, pl.ds(s * ch + half * zr, zr)])

  return k


def _tc_prep(d0, d1, x0):
  np_, d = x0.shape
  br = 1024

  def body(d0_ref, d1_ref, x0_ref, y_ref):
    r = lax.rsqrt(1.0 + d0_ref[:, :1] + d1_ref[:, :1])
    y_ref[...] = r * x0_ref[...]

  return pl.pallas_call(
      body,
      grid=(np_ // br,),
      in_specs=[
          pl.BlockSpec((br, 16), lambda i: (i, 0)),
          pl.BlockSpec((br, 16), lambda i: (i, 0)),
          pl.BlockSpec((br, d), lambda i: (i, 0)),
      ],
      out_specs=pl.BlockSpec((br, d), lambda i: (i, 0)),
      out_shape=jax.ShapeDtypeStruct((np_, d), jnp.float32),
  )(d0, d1, x0)


def _tc_combine(d0, d1, p0, p1, y, x0):
  np_, d = x0.shape
  br = 1024

  def body(d0_ref, d1_ref, p0_ref, p1_ref, y_ref, x0_ref, x_ref, yn_ref):
    r = lax.rsqrt(1.0 + d0_ref[:, :1] + d1_ref[:, :1])
    x = (1.0 - ALPHA) * r * (p0_ref[...] + p1_ref[...] + y_ref[...]) \
        + ALPHA * x0_ref[...]
    x_ref[...] = x
    yn_ref[...] = r * x

  row_spec = pl.BlockSpec((br, d), lambda i: (i, 0))
  deg_spec = pl.BlockSpec((br, 16), lambda i: (i, 0))
  return pl.pallas_call(
      body,
      grid=(np_ // br,),
      in_specs=[deg_spec, deg_spec, row_spec, row_spec, row_spec, row_spec],
      out_specs=[row_spec, row_spec],
      out_shape=[
          jax.ShapeDtypeStruct((np_, d), jnp.float32),
          jax.ShapeDtypeStruct((np_, d), jnp.float32),
      ],
  )(d0, d1, p0, p1, y, x0)


@jax.jit
def kernel(features, edge_index):
  n, d = features.shape
  e = edge_index.shape[1]

  np_ = ((n + 1 + 1023) // 1024) * 1024
  nb = -(-e // (NW * B))
  epad = NW * nb * B
  pad = epad - e

  src = jnp.concatenate(
      [edge_index[0], jnp.full((pad,), n, dtype=jnp.int32)]
  ).reshape(NW, nb, B)
  dst = jnp.concatenate(
      [edge_index[1], jnp.full((pad,), n, dtype=jnp.int32)]
  ).reshape(NW, nb, B)

  x0 = jnp.zeros((np_, d), jnp.float32).at[:n].set(features)
  ones16 = jnp.ones((B, 16), jnp.float32)
  zeros16 = jnp.zeros((np_ // NS, 16), jnp.float32)
  zerosd = jnp.zeros((np_ // NS // 2, d), jnp.float32)

  deg_p = _sc_degree_kernel(np_, nb)(dst, ones16, zeros16)
  d0, d1 = deg_p[0], deg_p[1]

  hop = _sc_hop_kernel(np_, d, nb)
  y = _tc_prep(d0, d1, x0)
  x = x0
  for _ in range(K):
    hp = hop(y, src, dst, zerosd)
    x, y = _tc_combine(d0, d1, hp[0], hp[1], y, x0)
  return x[:n]


# speedup vs baseline: 3.2396x; 3.2396x over previous
import functools

import jax
import jax.numpy as jnp
from jax import lax
from jax.experimental import pallas as pl
from jax.experimental.pallas import tpu as pltpu
from jax.experimental.pallas import tpu_sc as plsc

ALPHA = 0.1
K = 3
NC = 2
NS = 16
B = 128
NP = 2
QR = 2560


def _sc_hop_kernel(np_, d, nb2):
  half = np_ // NC
  qch = QR // NS
  mesh = plsc.VectorSubcoreMesh(core_axis_name="c", subcore_axis_name="s")

  @functools.partial(
      pl.kernel,
      out_type=jax.ShapeDtypeStruct((NC, half, d), jnp.float32),
      mesh=mesh,
      scratch_types=[
          pltpu.VMEM((nb2, B), jnp.int32),
          pltpu.VMEM((nb2, B), jnp.int32),
          pltpu.VMEM((nb2, B), jnp.int32),
          pltpu.VMEM((B, d), jnp.float32),
          pltpu.VMEM((QR // NS, d), jnp.float32),
          pltpu.VMEM_SHARED((QR + 8, d), jnp.float32),
          pltpu.SemaphoreType.DMA,
      ],
  )
  def k(y_hbm, src_hbm, dst_hbm, zeros_hbm, out_hbm, src_v, dst_v, ldst_v,
        rows_v, zbuf_v, accum, sem):
    c = lax.axis_index("c")
    s = lax.axis_index("s")
    pltpu.sync_copy(zeros_hbm, zbuf_v)
    pltpu.sync_copy(src_hbm.at[s], src_v)
    pltpu.sync_copy(dst_hbm.at[s], dst_v)

    for p in range(NP):
      base = c * half + p * QR

      def remap(j, carry):
        for kk in range(B // 16):
          sl = pl.ds(kk * 16, 16)
          v = dst_v[j, sl] - base
          ok = (v >= 0) & (v < QR)
          ldst_v[j, sl] = jnp.where(ok, v, QR)
        return carry

      lax.fori_loop(0, nb2, remap, 0)
      pltpu.sync_copy(zbuf_v, accum.at[pl.ds(s * qch, qch)])
      plsc.subcore_barrier()

      def body(j, carry):
        pltpu.async_copy(y_hbm.at[src_v.at[j]], rows_v, sem).wait()
        pltpu.sync_copy(rows_v, accum.at[ldst_v.at[j]], add=True)
        return carry

      lax.fori_loop(0, nb2, body, 0)
      plsc.subcore_barrier()
      pltpu.sync_copy(accum.at[pl.ds(s * qch, qch)], zbuf_v)
      pltpu.sync_copy(zbuf_v, out_hbm.at[c, pl.ds(p * QR + s * qch, qch)])
      if p + 1 < NP:
        plsc.subcore_barrier()
        pltpu.sync_copy(zeros_hbm, zbuf_v)

  return k


def _tc_prep(deg, x0):
  np_, d = x0.shape
  br = 1024

  def body(deg_ref, x0_ref, y_ref):
    r = lax.rsqrt(1.0 + deg_ref[...])
    y_ref[...] = r * x0_ref[...]

  spec = pl.BlockSpec((br, d), lambda i: (i, 0))
  return pl.pallas_call(
      body,
      grid=(np_ // br,),
      in_specs=[spec, spec],
      out_specs=spec,
      out_shape=jax.ShapeDtypeStruct((np_, d), jnp.float32),
  )(deg, x0)


def _tc_combine(deg, agg, y, x0):
  np_, d = x0.shape
  br = 1024

  def body(deg_ref, agg_ref, y_ref, x0_ref, x_ref, yn_ref):
    r = lax.rsqrt(1.0 + deg_ref[...])
    x = (1.0 - ALPHA) * r * (agg_ref[...] + y_ref[...]) + ALPHA * x0_ref[...]
    x_ref[...] = x
    yn_ref[...] = r * x

  spec = pl.BlockSpec((br, d), lambda i: (i, 0))
  return pl.pallas_call(
      body,
      grid=(np_ // br,),
      in_specs=[spec, spec, spec, spec],
      out_specs=[spec, spec],
      out_shape=[
          jax.ShapeDtypeStruct((np_, d), jnp.float32),
          jax.ShapeDtypeStruct((np_, d), jnp.float32),
      ],
  )(deg, agg, y, x0)


@jax.jit
def kernel(features, edge_index):
  n, d = features.shape
  e = edge_index.shape[1]

  np_ = ((n + 1 + 2047) // 2048) * 2048
  nb2 = -(-e // (NS * B))
  epad = NS * nb2 * B
  pad = epad - e

  src = jnp.concatenate(
      [edge_index[0], jnp.full((pad,), n, dtype=jnp.int32)]
  ).reshape(NS, nb2, B)
  dst = jnp.concatenate(
      [edge_index[1], jnp.full((pad,), n, dtype=jnp.int32)]
  ).reshape(NS, nb2, B)

  x0 = jnp.zeros((np_, d), jnp.float32).at[:n].set(features)
  onesf = jnp.ones((np_, d), jnp.float32)
  zerosb = jnp.zeros((QR // NS, d), jnp.float32)

  hop = _sc_hop_kernel(np_, d, nb2)

  deg = hop(onesf, src, dst, zerosb).reshape(np_, d)
  y = _tc_prep(deg, x0)
  x = x0
  for _ in range(K):
    agg = hop(y, src, dst, zerosb).reshape(np_, d)
    x, y = _tc_combine(deg, agg, y, x0)
  return x[:n]


# speedup vs baseline: 5.1197x; 1.5803x over previous
import functools

import jax
import jax.numpy as jnp
from jax import lax
from jax.experimental import pallas as pl
from jax.experimental.pallas import tpu as pltpu
from jax.experimental.pallas import tpu_sc as plsc

ALPHA = 0.1
K = 3
NC = 2
NS = 16
B = 128
NP = 1
QR = 5120


NBUF = 2


def _sc_hop_kernel(np_, d, nb2):
  half = np_ // NC
  qch = QR // NS
  ng = nb2 // NBUF
  mesh = plsc.VectorSubcoreMesh(core_axis_name="c", subcore_axis_name="s")

  @functools.partial(
      pl.kernel,
      out_type=jax.ShapeDtypeStruct((NC, half, d), jnp.float32),
      mesh=mesh,
      scratch_types=[
          pltpu.VMEM((nb2, B), jnp.int32),
          pltpu.VMEM((nb2, B), jnp.int32),
          pltpu.VMEM((NBUF, B), jnp.int32),
          *[pltpu.VMEM((B, d), jnp.float32) for _ in range(NBUF)],
          pltpu.VMEM((64, d), jnp.float32),
          pltpu.VMEM_SHARED((QR + 8, d), jnp.float32),
          *[pltpu.SemaphoreType.DMA for _ in range(2 * NBUF)],
      ],
  )
  def k(y_hbm, src_hbm, dst_hbm, zeros_hbm, out_hbm, src_v, gdst_v, scidx_v,
        *rest):
    rows = rest[:NBUF]
    zbuf_v = rest[NBUF]
    accum = rest[NBUF + 1]
    gsem = rest[NBUF + 2:NBUF + 2 + NBUF]
    ssem = rest[NBUF + 2 + NBUF:]
    c = lax.axis_index("c")
    s = lax.axis_index("s")
    pltpu.sync_copy(zeros_hbm, zbuf_v)
    pltpu.sync_copy(src_hbm.at[s], src_v)
    pltpu.sync_copy(dst_hbm.at[s], gdst_v)

    cbase = c * half

    def mklocal(j, carry):
      for kk in range(B // 16):
        sl = pl.ds(kk * 16, 16)
        gdst_v[j, sl] = gdst_v[j, sl] - cbase
      return carry

    lax.fori_loop(0, nb2, mklocal, 0)

    for p in range(NP):
      pbase = p * QR
      for z in range(qch // 64):
        pltpu.sync_copy(zbuf_v, accum.at[pl.ds(s * qch + z * 64, 64)])
      plsc.subcore_barrier()

      for b in range(NBUF):
        pltpu.async_copy(y_hbm.at[src_v.at[b]], rows[b], gsem[b])

      def group(g, carry):
        for b in range(NBUF):
          j = g * NBUF + b
          pltpu.make_async_copy(y_hbm.at[src_v.at[j]], rows[b],
                                gsem[b]).wait()
          for kk in range(B // 16):
            sl = pl.ds(kk * 16, 16)
            v = gdst_v[j, sl] - pbase
            ok = (v >= 0) & (v < QR)
            scidx_v[b, sl] = jnp.where(ok, v, QR)
          pltpu.async_copy(rows[b], accum.at[scidx_v.at[b]], ssem[b],
                           add=True)
        for b in range(NBUF):
          j = g * NBUF + b
          pltpu.make_async_copy(rows[b], accum.at[scidx_v.at[b]],
                                ssem[b]).wait()

          @pl.when(g < ng - 1)
          def _():
            jn = (g + 1) * NBUF + b
            pltpu.async_copy(y_hbm.at[src_v.at[jn]], rows[b], gsem[b])

        return carry

      lax.fori_loop(0, ng, group, 0)
      plsc.subcore_barrier()
      for z in range(qch // 64):
        pltpu.sync_copy(accum.at[pl.ds(s * qch + z * 64, 64)], zbuf_v)
        pltpu.sync_copy(zbuf_v,
                        out_hbm.at[c, pl.ds(p * QR + s * qch + z * 64, 64)])
      if p + 1 < NP:
        plsc.subcore_barrier()
        pltpu.sync_copy(zeros_hbm, zbuf_v)

  return k


def _tc_prep(deg, x0):
  np_, d = x0.shape
  br = 1024

  def body(deg_ref, x0_ref, y_ref):
    r = lax.rsqrt(1.0 + deg_ref[...])
    y_ref[...] = r * x0_ref[...]

  spec = pl.BlockSpec((br, d), lambda i: (i, 0))
  return pl.pallas_call(
      body,
      grid=(np_ // br,),
      in_specs=[spec, spec],
      out_specs=spec,
      out_shape=jax.ShapeDtypeStruct((np_, d), jnp.float32),
  )(deg, x0)


def _tc_combine(deg, agg, y, x0):
  np_, d = x0.shape
  br = 1024

  def body(deg_ref, agg_ref, y_ref, x0_ref, x_ref, yn_ref):
    r = lax.rsqrt(1.0 + deg_ref[...])
    x = (1.0 - ALPHA) * r * (agg_ref[...] + y_ref[...]) + ALPHA * x0_ref[...]
    x_ref[...] = x
    yn_ref[...] = r * x

  spec = pl.BlockSpec((br, d), lambda i: (i, 0))
  return pl.pallas_call(
      body,
      grid=(np_ // br,),
      in_specs=[spec, spec, spec, spec],
      out_specs=[spec, spec],
      out_shape=[
          jax.ShapeDtypeStruct((np_, d), jnp.float32),
          jax.ShapeDtypeStruct((np_, d), jnp.float32),
      ],
  )(deg, agg, y, x0)


@jax.jit
def kernel(features, edge_index):
  n, d = features.shape
  e = edge_index.shape[1]

  np_ = ((n + 1 + 2047) // 2048) * 2048
  nb2 = -(-e // (NS * B))
  nb2 = ((nb2 + NBUF - 1) // NBUF) * NBUF
  epad = NS * nb2 * B
  pad = epad - e

  src = jnp.concatenate(
      [edge_index[0], jnp.full((pad,), n, dtype=jnp.int32)]
  ).reshape(NS, nb2, B)
  dst = jnp.concatenate(
      [edge_index[1], jnp.full((pad,), n, dtype=jnp.int32)]
  ).reshape(NS, nb2, B)

  x0 = jnp.zeros((np_, d), jnp.float32).at[:n].set(features)
  onesf = jnp.ones((np_, d), jnp.float32)
  zerosb = jnp.zeros((64, d), jnp.float32)

  hop = _sc_hop_kernel(np_, d, nb2)

  deg = hop(onesf, src, dst, zerosb).reshape(np_, d)
  y = _tc_prep(deg, x0)
  x = x0
  for _ in range(K):
    agg = hop(y, src, dst, zerosb).reshape(np_, d)
    x, y = _tc_combine(deg, agg, y, x0)
  return x[:n]
